# Initial kernel scaffold; baseline (speedup 1.0000x reference)
import functools

import jax
import jax.numpy as jnp
from jax import lax
from jax.experimental import pallas as pl
from jax.experimental.pallas import tpu as pltpu
from jax.experimental.pallas import tpu_sc as plsc

MDIM = 4096
YLEN = 8192
NRES = 8
BLK = 512
NLANES = 16
NCORES = 2
NSUBCORES = 16
NWORKERS = NCORES * NSUBCORES
ROWS_PER_WORKER = MDIM // NWORKERS
ITEMS = NRES * (YLEN // BLK)
ITEMS_PER_SUBCORE = ITEMS // NSUBCORES


def _body(x_hbm, out_hbm, xv, buf, yv, sem):
    c = lax.axis_index("c")
    s = lax.axis_index("s")

    pltpu.sync_copy(x_hbm, xv)

    lanes = lax.iota(jnp.int32, 16)

    for k in range(ITEMS_PER_SUBCORE):
        item = s * ITEMS_PER_SUBCORE + k
        r = item // (YLEN // BLK)
        b = item % (YLEN // BLK)

        def chunk_body(j, _, r=r, b=b):
            pos = b * BLK + j * NLANES + lanes + r - (MDIM - 1)
            idx = jnp.minimum(jnp.abs(pos), MDIM - 1)
            buf[pl.ds(j * NLANES, NLANES)] = plsc.load_gather(xv, [idx])
            return 0

        lax.fori_loop(0, BLK // NLANES, chunk_body, 0)
        pltpu.sync_copy(buf, yv.at[pl.ds(r * YLEN + b * BLK, BLK)])

    plsc.subcore_barrier()

    row0 = (s * NCORES + c) * ROWS_PER_WORKER

    def row_copy(t):
        i = row0 + t
        o = (MDIM - 1) - i
        r = o % NRES
        q = o - r
        src = yv.at[pl.ds(r * YLEN + q, MDIM)]
        dst = out_hbm.at[pl.ds(i * MDIM, MDIM)]
        return pltpu.make_async_copy(src, dst, sem)

    def fire(t, _):
        row_copy(t).start()
        return 0

    def drain(t, _):
        row_copy(t).wait()
        return 0

    lax.fori_loop(0, ROWS_PER_WORKER, fire, 0)
    lax.fori_loop(0, ROWS_PER_WORKER, drain, 0)


_fill = functools.partial(
    pl.kernel,
    out_type=jax.ShapeDtypeStruct((MDIM * MDIM,), jnp.float32),
    mesh=plsc.VectorSubcoreMesh(core_axis_name="c", subcore_axis_name="s"),
    scratch_types=[
        pltpu.VMEM((MDIM,), jnp.float32),
        pltpu.VMEM((BLK,), jnp.float32),
        pltpu.VMEM_SHARED((NRES * YLEN,), jnp.float32),
        pltpu.SemaphoreType.DMA,
    ],
)(_body)


def kernel(input):
    x = input.reshape(-1)
    out_flat = _fill(x)
    return out_flat.reshape(1, MDIM, MDIM)


# speedup vs baseline: 1113.4297x; 1113.4297x over previous
import functools

import jax
import jax.numpy as jnp
from jax import lax
from jax.experimental import pallas as pl
from jax.experimental.pallas import tpu as pltpu
from jax.experimental.pallas import tpu_sc as plsc

MDIM = 4096
YLEN = 8192
NRES = 8
BLK = 512
NLANES = 16
NCORES = 2
NSUBCORES = 16
NWORKERS = NCORES * NSUBCORES
ROWS_PER_WORKER = MDIM // NWORKERS
ITEMS = NRES * (YLEN // BLK)
ITEMS_PER_SUBCORE = ITEMS // NSUBCORES


STRIP = 4224
CHUNKS = STRIP // NLANES
ROWS_PER_RES = ROWS_PER_WORKER // NRES


def _body(x_hbm, out_hbm, xv, yloc, sem):
    c = lax.axis_index("c")
    s = lax.axis_index("s")

    pltpu.sync_copy(x_hbm, xv)

    lanes = lax.iota(jnp.int32, 16)

    row0 = (s * NCORES + c) * ROWS_PER_WORKER
    o_lo = MDIM - ROWS_PER_WORKER - row0

    for r in range(NRES):
        def chunk_body(j, _, r=r):
            pos = o_lo + j * NLANES + lanes + r - (MDIM - 1)
            idx = jnp.minimum(jnp.abs(pos), MDIM - 1)
            off = pl.multiple_of(r * STRIP + j * NLANES, NLANES)
            yloc[pl.ds(off, NLANES)] = plsc.load_gather(xv, [idx])
            return 0

        lax.fori_loop(0, CHUNKS, chunk_body, 0)

        def fire(k, _, r=r):
            i = row0 + ROWS_PER_WORKER - 1 - r - NRES * k
            src = yloc.at[pl.ds(pl.multiple_of(r * STRIP + NRES * k, NRES), MDIM)]
            dst = out_hbm.at[pl.ds(pl.multiple_of(i * MDIM, MDIM), MDIM)]
            pltpu.make_async_copy(src, dst, sem).start()
            return 0

        lax.fori_loop(0, ROWS_PER_RES, fire, 0)

    def drain(t, _):
        src = yloc.at[pl.ds(0, MDIM)]
        dst = out_hbm.at[pl.ds(pl.multiple_of((row0 + t) * MDIM, MDIM), MDIM)]
        pltpu.make_async_copy(src, dst, sem).wait()
        return 0

    lax.fori_loop(0, ROWS_PER_WORKER, drain, 0)


_fill = functools.partial(
    pl.kernel,
    out_type=jax.ShapeDtypeStruct((MDIM * MDIM,), jnp.float32),
    mesh=plsc.VectorSubcoreMesh(core_axis_name="c", subcore_axis_name="s"),
    scratch_types=[
        pltpu.VMEM((MDIM,), jnp.float32),
        pltpu.VMEM((NRES * STRIP,), jnp.float32),
        pltpu.SemaphoreType.DMA,
    ],
    compiler_params=pltpu.CompilerParams(needs_layout_passes=False),
)(_body)


def kernel(input):
    x = input.reshape(-1)
    out_flat = _fill(x)
    return out_flat.reshape(1, MDIM, MDIM)
